# Initial kernel scaffold; baseline (speedup 1.0000x reference)
import jax
import jax.numpy as jnp
from jax.experimental import pallas as pl


def kernel(origins, directions, occ_grid):
    raise NotImplementedError("write your pallas kernel here")


# speedup vs baseline: 6.8375x; 6.8375x over previous
import functools

import jax
import jax.numpy as jnp
from jax import lax
from jax.experimental import pallas as pl
from jax.experimental.pallas import tpu as pltpu
from jax.experimental.pallas import tpu_sc as plsc

_AABB_MIN = -64.0
_AABB_MAX = 64.0
_GRID_RES = 128
_N_STEPS = 192
_RENDER_STEP = 1.0
_ALPHA_THRE = 0.01
_N_RAYS = 16384

_NC = 2
_NSUB = 16
_NW = _NC * _NSUB
_LANES = 16
_RAYS_PER_W = _N_RAYS // _NW
_CHUNK = 16
_CHUNKS_PER_W = _RAYS_PER_W // _CHUNK
_NSAMP = _N_RAYS * _N_STEPS
_WORDS = (_GRID_RES ** 3) // 32


def _sc_march(rays3, occ_words):
    mesh = plsc.VectorSubcoreMesh(
        core_axis_name="c", subcore_axis_name="s",
        num_cores=_NC, num_subcores=_NSUB)

    cbuf = _CHUNK * _N_STEPS

    @functools.partial(
        pl.kernel,
        out_type=(
            jax.ShapeDtypeStruct((_NSAMP * 3,), jnp.float32),
            jax.ShapeDtypeStruct((_NSAMP * 3,), jnp.float32),
            jax.ShapeDtypeStruct((_NSAMP,), jnp.float32),
            jax.ShapeDtypeStruct((_NSAMP,), jnp.float32),
            jax.ShapeDtypeStruct((_NSAMP,), jnp.int32),
        ),
        mesh=mesh,
        compiler_params=pltpu.CompilerParams(needs_layout_passes=False),
        scratch_types=[
            pltpu.VMEM((_WORDS,), jnp.int32),
            pltpu.VMEM((8, _LANES), jnp.float32),
            pltpu.VMEM((cbuf * 3,), jnp.float32),
            pltpu.VMEM((cbuf * 3,), jnp.float32),
            pltpu.VMEM((cbuf,), jnp.float32),
            pltpu.VMEM((cbuf,), jnp.float32),
            pltpu.VMEM((cbuf,), jnp.int32),
        ],
    )
    def march(rays_hbm, words_hbm, so_hbm, sd_hbm, st_hbm, en_hbm, ri_hbm,
              table_v, ray_v, so_v, sd_v, st_v, en_v, ri_v):
        cid = lax.axis_index("c")
        sid = lax.axis_index("s")
        wid = sid * _NC + cid

        pltpu.sync_copy(words_hbm, table_v)

        lanes = jnp.arange(_LANES, dtype=jnp.int32)
        lane_steps = lanes * _N_STEPS

        @pl.loop(0, _CHUNKS_PER_W)
        def chunk_loop(ci):
            chunk_id = wid * _CHUNKS_PER_W + ci
            r0 = chunk_id * _CHUNK
            pltpu.sync_copy(rays_hbm.at[chunk_id], ray_v)
            ox = ray_v[0]
            oy = ray_v[1]
            oz = ray_v[2]
            dx = ray_v[3]
            dy = ray_v[4]
            dz = ray_v[5]
            tnear = ray_v[6]
            tfar = ray_v[7]
            ray_ok = tfar > tnear
            rvec = r0 + lanes

            @pl.loop(0, _N_STEPS)
            def step_loop(s):
                s_f = s.astype(jnp.float32)
                t_start = tnear + s_f
                t_end = t_start + _RENDER_STEP
                t_mid = (t_start + t_end) * 0.5
                px = (ox + dx * t_mid) - _AABB_MIN
                py = (oy + dy * t_mid) - _AABB_MIN
                pz = (oz + dz * t_mid) - _AABB_MIN
                vx = jnp.minimum(jnp.maximum(px, 0.0), 127.0).astype(jnp.int32)
                vy = jnp.minimum(jnp.maximum(py, 0.0), 127.0).astype(jnp.int32)
                vz = jnp.minimum(jnp.maximum(pz, 0.0), 127.0).astype(jnp.int32)
                lin = (vx * 128 + vy) * 128 + vz
                word = lax.shift_right_logical(lin, 5)
                bit = lin & 31
                w = plsc.load_gather(table_v, [word])
                occ = (lax.shift_right_logical(w, bit) & 1) == 1
                keep = (t_end <= tfar) & ray_ok & occ
                keepf = jnp.where(keep, 1.0, 0.0).astype(jnp.float32)

                idx1 = lane_steps + s
                idx3 = idx1 * 3
                plsc.store_scatter(st_v, [idx1], t_start * keepf)
                plsc.store_scatter(en_v, [idx1], t_end * keepf)
                plsc.store_scatter(ri_v, [idx1],
                                   jnp.where(keep, rvec, 0))
                plsc.store_scatter(so_v, [idx3], ox * keepf)
                plsc.store_scatter(so_v, [idx3 + 1], oy * keepf)
                plsc.store_scatter(so_v, [idx3 + 2], oz * keepf)
                plsc.store_scatter(sd_v, [idx3], dx * keepf)
                plsc.store_scatter(sd_v, [idx3 + 1], dy * keepf)
                plsc.store_scatter(sd_v, [idx3 + 2], dz * keepf)

            base1 = pl.multiple_of(r0 * _N_STEPS, cbuf)
            base3 = pl.multiple_of(r0 * _N_STEPS * 3, cbuf * 3)
            pltpu.sync_copy(so_v, so_hbm.at[pl.ds(base3, cbuf * 3)])
            pltpu.sync_copy(sd_v, sd_hbm.at[pl.ds(base3, cbuf * 3)])
            pltpu.sync_copy(st_v, st_hbm.at[pl.ds(base1, cbuf)])
            pltpu.sync_copy(en_v, en_hbm.at[pl.ds(base1, cbuf)])
            pltpu.sync_copy(ri_v, ri_hbm.at[pl.ds(base1, cbuf)])

    return march(rays3, occ_words)


def kernel(origins, directions, occ_grid):
    safe_d = jnp.where(jnp.abs(directions) < 1e-10, 1e-10, directions)
    inv_d = 1.0 / safe_d
    t0 = (_AABB_MIN - origins) * inv_d
    t1 = (_AABB_MAX - origins) * inv_d
    t_near = jnp.maximum(jnp.max(jnp.minimum(t0, t1), axis=-1), 0.0)
    t_far = jnp.min(jnp.maximum(t0, t1), axis=-1)

    fields = jnp.stack([
        origins[:, 0], origins[:, 1], origins[:, 2],
        directions[:, 0], directions[:, 1], directions[:, 2],
        t_near, t_far,
    ], axis=0)
    rays3 = fields.reshape(8, _N_RAYS // _CHUNK, _CHUNK).transpose(1, 0, 2)

    occ_bool = (occ_grid.reshape(-1, 32) > _ALPHA_THRE).astype(jnp.uint32)
    occ_words = (occ_bool << jnp.arange(32, dtype=jnp.uint32)[None, :]
                 ).sum(axis=1).astype(jnp.int32)

    so_f, sd_f, st, en, ri = _sc_march(rays3, occ_words)

    s_origins = so_f.reshape(_NSAMP, 3)
    s_dirs = sd_f.reshape(_NSAMP, 3)
    starts_out = st.reshape(_NSAMP, 1)
    ends_out = en.reshape(_NSAMP, 1)
    pixel_area = jnp.zeros((_NSAMP, 1), jnp.float32)
    return (s_origins, s_dirs, starts_out, ends_out, pixel_area, ri)


# speedup vs baseline: 15.8706x; 2.3211x over previous
import functools

import jax
import jax.numpy as jnp
from jax import lax
from jax.experimental import pallas as pl
from jax.experimental.pallas import tpu as pltpu
from jax.experimental.pallas import tpu_sc as plsc

_AABB_MIN = -64.0
_AABB_MAX = 64.0
_GRID_RES = 128
_N_STEPS = 192
_RENDER_STEP = 1.0
_ALPHA_THRE = 0.01
_N_RAYS = 16384

_NC = 2
_NSUB = 16
_NW = _NC * _NSUB
_LANES = 16
_RAYS_PER_W = _N_RAYS // _NW
_CHUNK = 16
_CHUNKS_PER_W = _RAYS_PER_W // _CHUNK
_NSAMP = _N_RAYS * _N_STEPS
_WORDS = (_GRID_RES ** 3) // 32


def _sc_march(rays3, occ_words):
    mesh = plsc.VectorSubcoreMesh(
        core_axis_name="c", subcore_axis_name="s",
        num_cores=_NC, num_subcores=_NSUB)

    cbuf = _CHUNK * _N_STEPS

    @functools.partial(
        pl.kernel,
        out_type=(
            jax.ShapeDtypeStruct((_NSAMP,), jnp.float32),
            jax.ShapeDtypeStruct((_NSAMP,), jnp.float32),
            jax.ShapeDtypeStruct((_NSAMP,), jnp.int32),
        ),
        mesh=mesh,
        compiler_params=pltpu.CompilerParams(needs_layout_passes=False),
        scratch_types=[
            pltpu.VMEM((_WORDS,), jnp.int32),
            pltpu.VMEM((8, _LANES), jnp.float32),
            pltpu.VMEM((cbuf,), jnp.float32),
            pltpu.VMEM((cbuf,), jnp.float32),
            pltpu.VMEM((cbuf,), jnp.int32),
        ],
    )
    def march(rays_hbm, words_hbm, st_hbm, en_hbm, ri_hbm,
              table_v, ray_v, st_v, en_v, ri_v):
        cid = lax.axis_index("c")
        sid = lax.axis_index("s")
        wid = sid * _NC + cid

        pltpu.sync_copy(words_hbm, table_v)

        lanes = jnp.arange(_LANES, dtype=jnp.int32)
        lane_steps = lanes * _N_STEPS

        @pl.loop(0, _CHUNKS_PER_W)
        def chunk_loop(ci):
            chunk_id = wid * _CHUNKS_PER_W + ci
            r0 = chunk_id * _CHUNK
            pltpu.sync_copy(rays_hbm.at[chunk_id], ray_v)
            ox = ray_v[0]
            oy = ray_v[1]
            oz = ray_v[2]
            dx = ray_v[3]
            dy = ray_v[4]
            dz = ray_v[5]
            tnear = ray_v[6]
            tfar = ray_v[7]
            ray_ok = tfar > tnear
            rvec = r0 + lanes

            @pl.loop(0, _N_STEPS)
            def step_loop(s):
                s_f = s.astype(jnp.float32)
                t_start = tnear + s_f
                t_end = t_start + _RENDER_STEP
                t_mid = (t_start + t_end) * 0.5
                px = (ox + dx * t_mid) - _AABB_MIN
                py = (oy + dy * t_mid) - _AABB_MIN
                pz = (oz + dz * t_mid) - _AABB_MIN
                vx = jnp.minimum(jnp.maximum(px, 0.0), 127.0).astype(jnp.int32)
                vy = jnp.minimum(jnp.maximum(py, 0.0), 127.0).astype(jnp.int32)
                vz = jnp.minimum(jnp.maximum(pz, 0.0), 127.0).astype(jnp.int32)
                lin = (vx * 128 + vy) * 128 + vz
                word = lax.shift_right_logical(lin, 5)
                bit = lin & 31
                w = plsc.load_gather(table_v, [word])
                occ = (lax.shift_right_logical(w, bit) & 1) == 1
                keep = (t_end <= tfar) & ray_ok & occ
                keepf = jnp.where(keep, 1.0, 0.0).astype(jnp.float32)

                idx1 = lane_steps + s
                plsc.store_scatter(st_v, [idx1], t_start * keepf)
                plsc.store_scatter(en_v, [idx1], t_end * keepf)
                plsc.store_scatter(ri_v, [idx1],
                                   jnp.where(keep, rvec, 0))

            base1 = pl.multiple_of(r0 * _N_STEPS, cbuf)
            pltpu.sync_copy(st_v, st_hbm.at[pl.ds(base1, cbuf)])
            pltpu.sync_copy(en_v, en_hbm.at[pl.ds(base1, cbuf)])
            pltpu.sync_copy(ri_v, ri_hbm.at[pl.ds(base1, cbuf)])

    return march(rays3, occ_words)


def kernel(origins, directions, occ_grid):
    safe_d = jnp.where(jnp.abs(directions) < 1e-10, 1e-10, directions)
    inv_d = 1.0 / safe_d
    t0 = (_AABB_MIN - origins) * inv_d
    t1 = (_AABB_MAX - origins) * inv_d
    t_near = jnp.maximum(jnp.max(jnp.minimum(t0, t1), axis=-1), 0.0)
    t_far = jnp.min(jnp.maximum(t0, t1), axis=-1)

    fields = jnp.stack([
        origins[:, 0], origins[:, 1], origins[:, 2],
        directions[:, 0], directions[:, 1], directions[:, 2],
        t_near, t_far,
    ], axis=0)
    rays3 = fields.reshape(8, _N_RAYS // _CHUNK, _CHUNK).transpose(1, 0, 2)

    occ_bool = (occ_grid.reshape(-1, 32) > _ALPHA_THRE).astype(jnp.uint32)
    occ_words = (occ_bool << jnp.arange(32, dtype=jnp.uint32)[None, :]
                 ).sum(axis=1).astype(jnp.int32)

    st, en, ri = _sc_march(rays3, occ_words)

    keepf = (en.reshape(_N_RAYS, _N_STEPS) > 0.0).astype(jnp.float32)
    s_origins = (origins[:, None, :] * keepf[:, :, None]).reshape(_NSAMP, 3)
    s_dirs = (directions[:, None, :] * keepf[:, :, None]).reshape(_NSAMP, 3)
    starts_out = st.reshape(_NSAMP, 1)
    ends_out = en.reshape(_NSAMP, 1)
    pixel_area = jnp.zeros((_NSAMP, 1), jnp.float32)
    return (s_origins, s_dirs, starts_out, ends_out, pixel_area, ri)


# speedup vs baseline: 42.7821x; 2.6957x over previous
import functools

import jax
import jax.numpy as jnp
from jax import lax
from jax.experimental import pallas as pl
from jax.experimental.pallas import tpu as pltpu
from jax.experimental.pallas import tpu_sc as plsc

_AABB_MIN = -64.0
_AABB_MAX = 64.0
_GRID_RES = 128
_N_STEPS = 192
_RENDER_STEP = 1.0
_ALPHA_THRE = 0.01
_N_RAYS = 16384

_NC = 2
_NSUB = 16
_NW = _NC * _NSUB
_LANES = 16
_RAYS_PER_W = _N_RAYS // _NW
_CHUNK = 16
_CHUNKS_PER_W = _RAYS_PER_W // _CHUNK
_NSAMP = _N_RAYS * _N_STEPS
_WORDS = (_GRID_RES ** 3) // 32
_CBUF = _CHUNK * _N_STEPS
_FBUF = _CBUF * 4


def _sc_march(rays3, occ_words):
    mesh = plsc.VectorSubcoreMesh(
        core_axis_name="c", subcore_axis_name="s",
        num_cores=_NC, num_subcores=_NSUB)

    @functools.partial(
        pl.kernel,
        out_type=(
            jax.ShapeDtypeStruct((_NSAMP * 4,), jnp.float32),
            jax.ShapeDtypeStruct((_NSAMP * 4,), jnp.float32),
            jax.ShapeDtypeStruct((_NSAMP,), jnp.float32),
            jax.ShapeDtypeStruct((_NSAMP,), jnp.float32),
        ),
        mesh=mesh,
        compiler_params=pltpu.CompilerParams(needs_layout_passes=False),
        scratch_types=[
            pltpu.VMEM((_WORDS,), jnp.int32),
            [pltpu.VMEM((8, _LANES), jnp.float32)] * 2,
            [pltpu.VMEM((_FBUF,), jnp.float32)] * 2,
            [pltpu.VMEM((_FBUF,), jnp.float32)] * 2,
            [pltpu.VMEM((_CBUF,), jnp.float32)] * 2,
            [pltpu.VMEM((_CBUF,), jnp.float32)] * 2,
            [pltpu.SemaphoreType.DMA] * 2,
            [pltpu.SemaphoreType.DMA] * 2,
        ],
    )
    def march(rays_hbm, words_hbm, so_hbm, sd_hbm, st_hbm, en_hbm,
              table_v, ray_v, so_v, sd_v, st_v, en_v, osem, rsem):
        cid = lax.axis_index("c")
        sid = lax.axis_index("s")
        wid = sid * _NC + cid
        chunk0 = wid * _CHUNKS_PER_W

        pltpu.sync_copy(words_hbm, table_v)
        pltpu.sync_copy(rays_hbm.at[chunk0], ray_v[0])

        lanes = jnp.arange(_LANES, dtype=jnp.int32)
        lane_steps = lanes * _N_STEPS

        out_descs = [None, None]
        ray_desc = [None, None]

        for ci in range(_CHUNKS_PER_W):
            b = ci % 2
            chunk_id = chunk0 + ci
            r0 = chunk_id * _CHUNK
            if out_descs[b] is not None:
                for dsc in out_descs[b]:
                    dsc.wait()
            if ci + 1 < _CHUNKS_PER_W:
                nb = (ci + 1) % 2
                ray_desc[nb] = pltpu.async_copy(
                    rays_hbm.at[chunk_id + 1], ray_v[nb], rsem[nb])
            if ray_desc[b] is not None:
                ray_desc[b].wait()

            rv = ray_v[b]
            ox = rv[0]
            oy = rv[1]
            oz = rv[2]
            dx = rv[3]
            dy = rv[4]
            dz = rv[5]
            tnear = rv[6]
            tfar = rv[7]
            ray_ok = tfar > tnear

            so_b, sd_b, st_b, en_b = so_v[b], sd_v[b], st_v[b], en_v[b]

            @pl.loop(0, _N_STEPS)
            def step_loop(s):
                s_f = s.astype(jnp.float32)
                t_start = tnear + s_f
                t_end = t_start + _RENDER_STEP
                t_mid = (t_start + t_end) * 0.5
                px = (ox + dx * t_mid) - _AABB_MIN
                py = (oy + dy * t_mid) - _AABB_MIN
                pz = (oz + dz * t_mid) - _AABB_MIN
                vx = jnp.minimum(jnp.maximum(px, 0.0), 127.0).astype(jnp.int32)
                vy = jnp.minimum(jnp.maximum(py, 0.0), 127.0).astype(jnp.int32)
                vz = jnp.minimum(jnp.maximum(pz, 0.0), 127.0).astype(jnp.int32)
                lin = (vx * 128 + vy) * 128 + vz
                word = lax.shift_right_logical(lin, 5)
                bit = lin & 31
                w = plsc.load_gather(table_v, [word])
                occ = (lax.shift_right_logical(w, bit) & 1) == 1
                keep = (t_end <= tfar) & ray_ok & occ
                keepf = jnp.where(keep, 1.0, 0.0).astype(jnp.float32)

                idx1 = lane_steps + s
                fx = (lax.shift_right_logical(idx1, 7) * 512) + (idx1 & 127)
                plsc.store_scatter(st_b, [idx1], t_start * keepf)
                plsc.store_scatter(en_b, [idx1], t_end * keepf)
                plsc.store_scatter(so_b, [fx], ox * keepf)
                plsc.store_scatter(so_b, [fx + 128], oy * keepf)
                plsc.store_scatter(so_b, [fx + 256], oz * keepf)
                plsc.store_scatter(sd_b, [fx], dx * keepf)
                plsc.store_scatter(sd_b, [fx + 128], dy * keepf)
                plsc.store_scatter(sd_b, [fx + 256], dz * keepf)

            base1 = pl.multiple_of(r0 * _N_STEPS, _CBUF)
            base4 = pl.multiple_of(r0 * _N_STEPS * 4, _FBUF)
            out_descs[b] = [
                pltpu.async_copy(so_b, so_hbm.at[pl.ds(base4, _FBUF)], osem[b]),
                pltpu.async_copy(sd_b, sd_hbm.at[pl.ds(base4, _FBUF)], osem[b]),
                pltpu.async_copy(st_b, st_hbm.at[pl.ds(base1, _CBUF)], osem[b]),
                pltpu.async_copy(en_b, en_hbm.at[pl.ds(base1, _CBUF)], osem[b]),
            ]

        for descs in out_descs:
            if descs is not None:
                for dsc in descs:
                    dsc.wait()

    return march(rays3, occ_words)


def kernel(origins, directions, occ_grid):
    safe_d = jnp.where(jnp.abs(directions) < 1e-10, 1e-10, directions)
    inv_d = 1.0 / safe_d
    t0 = (_AABB_MIN - origins) * inv_d
    t1 = (_AABB_MAX - origins) * inv_d
    t_near = jnp.maximum(jnp.max(jnp.minimum(t0, t1), axis=-1), 0.0)
    t_far = jnp.min(jnp.maximum(t0, t1), axis=-1)

    fields = jnp.stack([
        origins[:, 0], origins[:, 1], origins[:, 2],
        directions[:, 0], directions[:, 1], directions[:, 2],
        t_near, t_far,
    ], axis=0)
    rays3 = fields.reshape(8, _N_RAYS // _CHUNK, _CHUNK).transpose(1, 0, 2)

    occ_bool = (occ_grid.reshape(-1, 32) > _ALPHA_THRE).astype(jnp.uint32)
    occ_words = (occ_bool << jnp.arange(32, dtype=jnp.uint32)[None, :]
                 ).sum(axis=1).astype(jnp.int32)

    so_f, sd_f, st, en = _sc_march(rays3, occ_words)

    s_origins = (so_f.reshape(_NSAMP // 128, 4, 128)[:, :3, :]
                 .transpose(0, 2, 1).reshape(_NSAMP, 3))
    s_dirs = (sd_f.reshape(_NSAMP // 128, 4, 128)[:, :3, :]
              .transpose(0, 2, 1).reshape(_NSAMP, 3))
    starts_out = st.reshape(_NSAMP, 1)
    ends_out = en.reshape(_NSAMP, 1)
    ray_ids = jnp.broadcast_to(
        jnp.arange(_N_RAYS, dtype=jnp.int32)[:, None],
        (_N_RAYS, _N_STEPS)).reshape(-1)
    ri = jnp.where(en > 0.0, ray_ids, 0)
    pixel_area = jnp.zeros((_NSAMP, 1), jnp.float32)
    return (s_origins, s_dirs, starts_out, ends_out, pixel_area, ri)


# speedup vs baseline: 99.1511x; 2.3176x over previous
import functools

import jax
import jax.numpy as jnp
from jax import lax
from jax.experimental import pallas as pl
from jax.experimental.pallas import tpu as pltpu
from jax.experimental.pallas import tpu_sc as plsc

_AABB_MIN = -64.0
_AABB_MAX = 64.0
_GRID_RES = 128
_N_STEPS = 192
_RENDER_STEP = 1.0
_ALPHA_THRE = 0.01
_N_RAYS = 16384

_NC = 2
_NSUB = 16
_NW = _NC * _NSUB
_LANES = 16
_RAYS_PER_W = _N_RAYS // _NW
_CHUNK = 16
_CHUNKS_PER_W = _RAYS_PER_W // _CHUNK
_NVEC = _N_STEPS // _LANES
_NSAMP = _N_RAYS * _N_STEPS
_WORDS = (_GRID_RES ** 3) // 32
_CBUF = _CHUNK * _N_STEPS
_FBUF = _CBUF * 4


def _sc_march(rays3, occ_words):
    mesh = plsc.VectorSubcoreMesh(
        core_axis_name="c", subcore_axis_name="s",
        num_cores=_NC, num_subcores=_NSUB)

    @functools.partial(
        pl.kernel,
        out_type=(
            jax.ShapeDtypeStruct((_NSAMP * 4,), jnp.float32),
            jax.ShapeDtypeStruct((_NSAMP * 4,), jnp.float32),
            jax.ShapeDtypeStruct((_NSAMP,), jnp.float32),
            jax.ShapeDtypeStruct((_NSAMP,), jnp.float32),
        ),
        mesh=mesh,
        compiler_params=pltpu.CompilerParams(needs_layout_passes=False),
        scratch_types=[
            pltpu.VMEM((_WORDS,), jnp.int32),
            [pltpu.VMEM((8, _CHUNK), jnp.float32)] * 2,
            [pltpu.VMEM((_FBUF,), jnp.float32)] * 2,
            [pltpu.VMEM((_FBUF,), jnp.float32)] * 2,
            [pltpu.VMEM((_CBUF,), jnp.float32)] * 2,
            [pltpu.VMEM((_CBUF,), jnp.float32)] * 2,
            [pltpu.SemaphoreType.DMA] * 2,
            [pltpu.SemaphoreType.DMA] * 2,
        ],
    )
    def march(rays_hbm, words_hbm, so_hbm, sd_hbm, st_hbm, en_hbm,
              table_v, ray_v, so_v, sd_v, st_v, en_v, osem, rsem):
        cid = lax.axis_index("c")
        sid = lax.axis_index("s")
        wid = sid * _NC + cid
        chunk0 = wid * _CHUNKS_PER_W

        pltpu.sync_copy(words_hbm, table_v)
        pltpu.sync_copy(rays_hbm.at[chunk0], ray_v[0])

        zf = jnp.zeros((_LANES,), jnp.float32)
        neg_inf = zf - 3.0e38
        iota_f = jnp.arange(_LANES, dtype=jnp.int32).astype(jnp.float32)
        svecs = [iota_f + float(k * _LANES) for k in range(_NVEC)]

        def out_copies(chunk_id, half, enqueue):
            r0 = chunk_id * _CHUNK
            base1 = pl.multiple_of(r0 * _N_STEPS, _CBUF)
            base4 = pl.multiple_of(r0 * _N_STEPS * 4, _FBUF)
            mk = pltpu.async_copy if enqueue else pltpu.make_async_copy
            return [
                mk(so_v[half], so_hbm.at[pl.ds(base4, _FBUF)], osem[half]),
                mk(sd_v[half], sd_hbm.at[pl.ds(base4, _FBUF)], osem[half]),
                mk(st_v[half], st_hbm.at[pl.ds(base1, _CBUF)], osem[half]),
                mk(en_v[half], en_hbm.at[pl.ds(base1, _CBUF)], osem[half]),
            ]

        @pl.loop(0, _CHUNKS_PER_W // 2)
        def pair_loop(pi):
            for half in range(2):
                ci = pi * 2 + half
                chunk_id = chunk0 + ci

                @pl.when(ci >= 2)
                def _wait_out():
                    for dsc in out_copies(chunk_id - 2, half, False):
                        dsc.wait()

                @pl.when(ci <= _CHUNKS_PER_W - 2)
                def _prefetch():
                    pltpu.async_copy(
                        rays_hbm.at[chunk_id + 1], ray_v[1 - half],
                        rsem[1 - half])

                @pl.when(ci >= 1)
                def _wait_ray():
                    pltpu.make_async_copy(
                        rays_hbm.at[chunk_id], ray_v[half],
                        rsem[half]).wait()

                rv = ray_v[half]
                so_b, sd_b = so_v[half], sd_v[half]
                st_b, en_b = st_v[half], en_v[half]
                rows = [rv[k] for k in range(8)]

                @pl.loop(0, _CHUNK)
                def ray_loop(r):
                    ridx = jnp.full((_LANES,), r, dtype=jnp.int32)

                    def bc(k):
                        return lax.gather(
                            rows[k], ridx[:, None],
                            lax.GatherDimensionNumbers(
                                offset_dims=(),
                                collapsed_slice_dims=(0,),
                                start_index_map=(0,)),
                            (1,),
                            mode=lax.GatherScatterMode.PROMISE_IN_BOUNDS)
                    o0 = bc(0)
                    o1 = bc(1)
                    o2 = bc(2)
                    dxv = bc(3)
                    dyv = bc(4)
                    dzv = bc(5)
                    tnv = bc(6)
                    tfv = bc(7)
                    tfev = jnp.where(tfv > tnv, tfv, neg_inf)

                    off0 = r * _N_STEPS
                    for k in range(_NVEC):
                        t_start = tnv + svecs[k]
                        t_end = t_start + _RENDER_STEP
                        t_mid = (t_start + t_end) * 0.5
                        px = (o0 + dxv * t_mid) - _AABB_MIN
                        py = (o1 + dyv * t_mid) - _AABB_MIN
                        pz = (o2 + dzv * t_mid) - _AABB_MIN
                        vx = jnp.minimum(jnp.maximum(px, 0.0), 127.0)
                        vy = jnp.minimum(jnp.maximum(py, 0.0), 127.0)
                        vz = jnp.minimum(jnp.maximum(pz, 0.0), 127.0)
                        lin = (vx.astype(jnp.int32) * 128
                               + vy.astype(jnp.int32)) * 128 \
                            + vz.astype(jnp.int32)
                        word = lax.shift_right_logical(lin, 5)
                        bit = lin & 31
                        w = plsc.load_gather(table_v, [word])
                        occ = (lax.shift_right_logical(w, bit) & 1) == 1
                        keep = (t_end <= tfev) & occ
                        keepf = jnp.where(keep, 1.0, 0.0)

                        off = off0 + k * _LANES
                        fx = (lax.shift_right_logical(off, 7) * 512) \
                            + (off & 127)
                        st_b[pl.ds(off, _LANES)] = t_start * keepf
                        en_b[pl.ds(off, _LANES)] = t_end * keepf
                        so_b[pl.ds(fx, _LANES)] = o0 * keepf
                        so_b[pl.ds(fx + 128, _LANES)] = o1 * keepf
                        so_b[pl.ds(fx + 256, _LANES)] = o2 * keepf
                        sd_b[pl.ds(fx, _LANES)] = dxv * keepf
                        sd_b[pl.ds(fx + 128, _LANES)] = dyv * keepf
                        sd_b[pl.ds(fx + 256, _LANES)] = dzv * keepf

                out_copies(chunk_id, half, True)

        for half, ci in ((0, _CHUNKS_PER_W - 2), (1, _CHUNKS_PER_W - 1)):
            for dsc in out_copies(chunk0 + ci, half, False):
                dsc.wait()

    return march(rays3, occ_words)


def kernel(origins, directions, occ_grid):
    safe_d = jnp.where(jnp.abs(directions) < 1e-10, 1e-10, directions)
    inv_d = 1.0 / safe_d
    t0 = (_AABB_MIN - origins) * inv_d
    t1 = (_AABB_MAX - origins) * inv_d
    t_near = jnp.maximum(jnp.max(jnp.minimum(t0, t1), axis=-1), 0.0)
    t_far = jnp.min(jnp.maximum(t0, t1), axis=-1)

    fields = jnp.stack([
        origins[:, 0], origins[:, 1], origins[:, 2],
        directions[:, 0], directions[:, 1], directions[:, 2],
        t_near, t_far,
    ], axis=0)
    rays3 = fields.reshape(8, _N_RAYS // _CHUNK, _CHUNK).transpose(1, 0, 2)

    occ_bool = (occ_grid.reshape(-1, 32) > _ALPHA_THRE).astype(jnp.uint32)
    occ_words = (occ_bool << jnp.arange(32, dtype=jnp.uint32)[None, :]
                 ).sum(axis=1).astype(jnp.int32)

    so_f, sd_f, st, en = _sc_march(rays3, occ_words)

    s_origins = (so_f.reshape(_NSAMP // 128, 4, 128)[:, :3, :]
                 .transpose(0, 2, 1).reshape(_NSAMP, 3))
    s_dirs = (sd_f.reshape(_NSAMP // 128, 4, 128)[:, :3, :]
              .transpose(0, 2, 1).reshape(_NSAMP, 3))
    starts_out = st.reshape(_NSAMP, 1)
    ends_out = en.reshape(_NSAMP, 1)
    ray_ids = jnp.broadcast_to(
        jnp.arange(_N_RAYS, dtype=jnp.int32)[:, None],
        (_N_RAYS, _N_STEPS)).reshape(-1)
    ri = jnp.where(en > 0.0, ray_ids, 0)
    pixel_area = jnp.zeros((_NSAMP, 1), jnp.float32)
    return (s_origins, s_dirs, starts_out, ends_out, pixel_area, ri)
